# Initial kernel scaffold; baseline (speedup 1.0000x reference)
import jax
import jax.numpy as jnp
from jax.experimental import pallas as pl


def kernel(seg_map, label_with_bg):
    raise NotImplementedError("write your pallas kernel here")


# speedup vs baseline: 191.8975x; 191.8975x over previous
import numpy as np

import jax
import jax.numpy as jnp
from jax.experimental import pallas as pl
from jax.experimental.pallas import tpu as pltpu

_BETA = 1000.0
_N, _C, _H, _W = 4, 21, 512, 512
_BR = 64
_NRB = _H // _BR

_GX = np.array([[2.0, 1.0, 1e-06, -1.0, -2.0],
                [3.0, 2.0, 1e-06, -2.0, -3.0],
                [4.0, 3.0, 0.0, -3.0, -4.0],
                [3.0, 2.0, 1e-06, -2.0, -3.0],
                [2.0, 1.0, 1e-06, -1.0, -2.0]], dtype=np.float32)
_GY = np.array([[2.0, 3.0, 4.0, 3.0, 2.0],
                [1.0, 2.0, 3.0, 2.0, 1.0],
                [1e-06, 1e-06, 1e-06, 1e-06, 1e-06],
                [-1.0, -2.0, -3.0, -2.0, -1.0],
                [-2.0, -3.0, -4.0, -3.0, -2.0]], dtype=np.float32)


def _body(lab_ref, seg_ref, out_ref, s_buf):
    n_idx = pl.program_id(0)
    rb = pl.program_id(1)

    x = seg_ref[0]
    m = jnp.max(x, axis=0)
    e = jnp.exp((x - m[None, :, :]) * _BETA)
    denom = jnp.sum(e, axis=0)
    wv = lab_ref[0][:, :, None]
    num = jnp.sum(e * wv, axis=0)
    s_buf[pl.ds(rb * _BR, _BR), :] = num / denom

    @pl.when(rb == _NRB - 1)
    def _conv_and_reduce():
        s = s_buf[:, :]
        zr = jnp.zeros((2, _W), jnp.float32)
        p = jnp.concatenate([zr, s, zr], axis=0)
        zc = jnp.zeros((_H + 4, 2), jnp.float32)
        p = jnp.concatenate([zc, p, zc], axis=1)
        gx = jnp.zeros((_H, _W), jnp.float32)
        gy = jnp.zeros((_H, _W), jnp.float32)
        for dy in range(5):
            for dx in range(5):
                tap = p[dy:dy + _H, dx:dx + _W]
                cx = float(_GX[dy, dx])
                cy = float(_GY[dy, dx])
                if cx != 0.0:
                    gx = gx + cx * tap
                if cy != 0.0:
                    gy = gy + cy * tap
        mag = jnp.sqrt(gx * gx + gy * gy + 1e-08)
        part = jnp.sum(mag)

        @pl.when(n_idx == 0)
        def _init():
            out_ref[:, :] = part[None, None]

        @pl.when(n_idx > 0)
        def _acc():
            out_ref[:, :] += part[None, None]


def kernel(seg_map, label_with_bg):
    n, c, h, w = seg_map.shape
    wz = label_with_bg.at[:, 0].set(0.0).reshape(n, c, 1)

    out = pl.pallas_call(
        _body,
        grid=(n, _NRB),
        in_specs=[
            pl.BlockSpec((1, c, 1), lambda i, j: (i, 0, 0)),
            pl.BlockSpec((1, c, _BR, w), lambda i, j: (i, 0, j, 0)),
        ],
        out_specs=pl.BlockSpec((1, 1), lambda i, j: (0, 0)),
        out_shape=jax.ShapeDtypeStruct((1, 1), jnp.float32),
        scratch_shapes=[pltpu.VMEM((_H, _W), jnp.float32)],
    )(wz, seg_map)
    return out[0, 0] / jnp.float32(n * h * w)


# speedup vs baseline: 305.7602x; 1.5934x over previous
import numpy as np

import jax
import jax.numpy as jnp
from jax.experimental import pallas as pl
from jax.experimental.pallas import tpu as pltpu

_BETA = 1000.0
_LOG2E = 1.4426950408889634
_N, _C, _H, _W = 4, 21, 512, 512
_BR = 64
_NRB = _H // _BR

_V0 = (2.0, 3.0, 4.0, 3.0, 2.0)
_V1 = (1.0, 2.0, 3.0, 2.0, 1.0)
_EPS = 1e-06


def _shift_cols(v, d, w):
    if d == 0:
        return v
    rows = v.shape[0]
    z = jnp.zeros((rows, abs(d)), jnp.float32)
    if d > 0:
        return jnp.concatenate([v[:, d:], z], axis=1)
    return jnp.concatenate([z, v[:, :d]], axis=1)


def _body(lab_ref, seg_ref, out_ref, s_buf):
    n_idx = pl.program_id(0)
    rb = pl.program_id(1)

    m = seg_ref[0, 0]
    for c in range(1, _C):
        m = jnp.maximum(m, seg_ref[0, c])
    scale = jnp.float32(_BETA * _LOG2E)
    wv = lab_ref[0]
    denom = jnp.exp2((seg_ref[0, 0] - m) * scale)
    num = None
    for c in range(1, _C):
        ec = jnp.exp2((seg_ref[0, c] - m) * scale)
        denom = denom + ec
        term = ec * wv[c][None]
        num = term if num is None else num + term
    s_buf[pl.ds(rb * _BR, _BR), :] = num / denom

    @pl.when(rb == _NRB - 1)
    def _conv_and_reduce():
        zr = jnp.zeros((2, _W), jnp.float32)
        s0 = jnp.concatenate([zr, s_buf[:, :], zr], axis=0)
        tm2 = _shift_cols(s0, -2, _W)
        tm1 = _shift_cols(s0, -1, _W)
        tp1 = _shift_cols(s0, 1, _W)
        tp2 = _shift_cols(s0, 2, _W)

        d0 = tm2 - tp2
        d1 = tm1 - tp1
        gx = jnp.zeros((_H, _W), jnp.float32)
        for dy in range(5):
            gx = gx + _V0[dy] * d0[dy:dy + _H, :]
        for dy in range(5):
            gx = gx + _V1[dy] * d1[dy:dy + _H, :]
        ex = (s0[0:_H, :] + s0[1:_H + 1, :]
              + s0[3:_H + 3, :] + s0[4:_H + 4, :])
        gx = gx + _EPS * ex

        gy = jnp.zeros((_H, _W), jnp.float32)
        rsum = jnp.zeros((_H, _W), jnp.float32)
        for d, t in ((-2, tm2), (-1, tm1), (0, s0), (1, tp1), (2, tp2)):
            p_d = t[0:_H, :] - t[4:_H + 4, :]
            q_d = t[1:_H + 1, :] - t[3:_H + 3, :]
            gy = gy + _V0[d + 2] * p_d + _V1[d + 2] * q_d
            rsum = rsum + t[2:_H + 2, :]
        gy = gy + _EPS * rsum

        mag = jnp.sqrt(gx * gx + gy * gy + 1e-08)
        part = jnp.sum(mag)

        @pl.when(n_idx == 0)
        def _init():
            out_ref[:, :] = part[None, None]

        @pl.when(n_idx > 0)
        def _acc():
            out_ref[:, :] += part[None, None]


def kernel(seg_map, label_with_bg):
    n, c, h, w = seg_map.shape
    wz = label_with_bg.at[:, 0].set(0.0).reshape(n, c, 1)

    out = pl.pallas_call(
        _body,
        grid=(n, _NRB),
        in_specs=[
            pl.BlockSpec((1, c, 1), lambda i, j: (i, 0, 0)),
            pl.BlockSpec((1, c, _BR, w), lambda i, j: (i, 0, j, 0)),
        ],
        out_specs=pl.BlockSpec((1, 1), lambda i, j: (0, 0)),
        out_shape=jax.ShapeDtypeStruct((1, 1), jnp.float32),
        scratch_shapes=[pltpu.VMEM((_H, _W), jnp.float32)],
    )(wz, seg_map)
    return out[0, 0] / jnp.float32(n * h * w)
